# baseline (speedup 1.0000x reference)
import functools

import jax
import jax.numpy as jnp
from jax import lax
from jax.experimental import pallas as pl
from jax.experimental.pallas import tpu as pltpu
from jax.experimental.pallas import tpu_sc as plsc

BATCH = 16384
EMBED_DIM = 16
RPR = 128 // EMBED_DIM
VPACK = 1000000 // RPR
NC = 2
NS = 16
NW = NC * NS
BPW = BATCH // NW
HB = BPW // 2
IDXC = 128
L = 16


def _sc_body(uid_hbm, pid_hbm, nid_hbm, uemb_hbm, iemb_hbm, ubias_hbm,
             ibias_hbm, gb_hbm, pos_out, neg_out,
             uid_v, pid_v, nid_v, uf_v, pf_v, nf_v, uhi_v, phi_v, nhi_v,
             ulo_v, plo_v, nlo_v, ue_v, iep_v, ien_v, ub_v, ibp_v, ibn_v,
             gb_v, outp_v, outn_v, sem):
    wid = lax.axis_index("s") * NC + lax.axis_index("c")

    pltpu.sync_copy(uid_hbm.at[wid], uid_v)
    pltpu.sync_copy(pid_hbm.at[wid], pid_v)
    pltpu.sync_copy(nid_hbm.at[wid], nid_v)
    pltpu.sync_copy(gb_hbm, gb_v)

    def split_body(c, carry):
        j = c // 8
        co = (c % 8) * L
        s = pl.ds(c * L, L)
        for idv, fv, hiv, lov in (
                (uid_v, uf_v, uhi_v, ulo_v),
                (pid_v, pf_v, phi_v, plo_v),
                (nid_v, nf_v, nhi_v, nlo_v)):
            iv = idv[j, pl.ds(co, L)]
            fv[s] = iv
            hiv[s] = iv >> 3
            lov[s] = (iv & (RPR - 1)) << 4
        return carry

    lax.fori_loop(0, BPW // L, split_body, 0)

    bias_copies = [
        pltpu.async_copy(ubias_hbm.at[uf_v], ub_v, sem),
        pltpu.async_copy(ibias_hbm.at[pf_v], ibp_v, sem),
        pltpu.async_copy(ibias_hbm.at[nf_v], ibn_v, sem),
    ]

    lane = lax.iota(jnp.int32, L)
    col15 = jnp.full((L,), EMBED_DIM - 1, jnp.int32)

    def gather_half(h):
        copies = []
        for j in range(HB // IDXC):
            src = pl.ds(h * HB + j * IDXC, IDXC)
            dst = pl.ds(j * IDXC, IDXC)
            copies.append(pltpu.async_copy(
                uemb_hbm.at[uhi_v.at[src]], ue_v.at[dst], sem))
            copies.append(pltpu.async_copy(
                iemb_hbm.at[phi_v.at[src]], iep_v.at[dst], sem))
            copies.append(pltpu.async_copy(
                iemb_hbm.at[nhi_v.at[src]], ien_v.at[dst], sem))
        return copies

    def compute_half(h):
        def dot_body(c, carry):
            s = pl.ds(h * HB + c * L, L)
            uos = ulo_v[s]
            pos_ = plo_v[s]
            nos = nlo_v[s]
            for k in range(L):
                e = c * L + k
                ue = ue_v[e, pl.ds(uos[k], L)]
                iep_v[e, pl.ds(0, L)] = plsc.cumsum(
                    ue * iep_v[e, pl.ds(pos_[k], L)])
                ien_v[e, pl.ds(0, L)] = plsc.cumsum(
                    ue * ien_v[e, pl.ds(nos[k], L)])
            return carry

        lax.fori_loop(0, HB // L, dot_body, 0)

        gb = gb_v[...]

        def epi_body(c, carry):
            cg = h * (HB // L) + c
            jo = cg // 8
            co = (cg % 8) * L
            s = pl.ds(cg * L, L)
            rows = lane + (c * L)
            dp = plsc.load_gather(iep_v, [rows, col15])
            dn = plsc.load_gather(ien_v, [rows, col15])
            ub = ub_v[s] + gb
            outp_v[jo, pl.ds(co, L)] = dp + (ub + ibp_v[s])
            outn_v[jo, pl.ds(co, L)] = dn + (ub + ibn_v[s])
            return carry

        lax.fori_loop(0, HB // L, epi_body, 0)

    copies0 = gather_half(0)
    for c in bias_copies:
        c.wait()
    for c in copies0:
        c.wait()
    compute_half(0)
    copies1 = gather_half(1)
    for c in copies1:
        c.wait()
    compute_half(1)

    pltpu.sync_copy(outp_v, pos_out.at[wid])
    pltpu.sync_copy(outn_v, neg_out.at[wid])


_sc_call = pl.kernel(
    _sc_body,
    out_type=(
        jax.ShapeDtypeStruct((NW, BPW // IDXC, IDXC), jnp.float32),
        jax.ShapeDtypeStruct((NW, BPW // IDXC, IDXC), jnp.float32),
    ),
    mesh=plsc.VectorSubcoreMesh(core_axis_name="c", subcore_axis_name="s",
                                num_cores=NC, num_subcores=NS),
    compiler_params=pltpu.CompilerParams(needs_layout_passes=False),
    scratch_types=[
        pltpu.VMEM((BPW // IDXC, IDXC), jnp.int32),
        pltpu.VMEM((BPW // IDXC, IDXC), jnp.int32),
        pltpu.VMEM((BPW // IDXC, IDXC), jnp.int32),
        pltpu.VMEM((BPW,), jnp.int32),
        pltpu.VMEM((BPW,), jnp.int32),
        pltpu.VMEM((BPW,), jnp.int32),
        pltpu.VMEM((BPW,), jnp.int32),
        pltpu.VMEM((BPW,), jnp.int32),
        pltpu.VMEM((BPW,), jnp.int32),
        pltpu.VMEM((BPW,), jnp.int32),
        pltpu.VMEM((BPW,), jnp.int32),
        pltpu.VMEM((BPW,), jnp.int32),
        pltpu.VMEM((HB, 128), jnp.float32),
        pltpu.VMEM((HB, 128), jnp.float32),
        pltpu.VMEM((HB, 128), jnp.float32),
        pltpu.VMEM((BPW,), jnp.float32),
        pltpu.VMEM((BPW,), jnp.float32),
        pltpu.VMEM((BPW,), jnp.float32),
        pltpu.VMEM((L,), jnp.float32),
        pltpu.VMEM((BPW // IDXC, IDXC), jnp.float32),
        pltpu.VMEM((BPW // IDXC, IDXC), jnp.float32),
        pltpu.SemaphoreType.DMA,
    ],
)


def kernel(user_ids, pos_item_ids, neg_item_ids, user_emb, item_emb,
           user_bias, item_bias, global_bias):
    uid = user_ids.astype(jnp.int32).reshape(NW, BPW // IDXC, IDXC)
    pid = pos_item_ids.astype(jnp.int32).reshape(NW, BPW // IDXC, IDXC)
    nid = neg_item_ids.astype(jnp.int32).reshape(NW, BPW // IDXC, IDXC)
    uep = user_emb.reshape(VPACK, 128)
    iep = item_emb.reshape(VPACK, 128)
    ub = jnp.sum(user_bias, axis=1)
    ib = jnp.sum(item_bias, axis=1)
    gb = jnp.broadcast_to(global_bias, (L,))
    pos, neg = _sc_call(uid, pid, nid, uep, iep, ub, ib, gb)
    return pos.reshape(BATCH), neg.reshape(BATCH)


# speedup vs baseline: 1.0260x; 1.0260x over previous
import functools

import jax
import jax.numpy as jnp
from jax import lax
from jax.experimental import pallas as pl
from jax.experimental.pallas import tpu as pltpu
from jax.experimental.pallas import tpu_sc as plsc

BATCH = 16384
EMBED_DIM = 16
NC = 2
NS = 16
NW = NC * NS
BPW = BATCH // NW
IDXC = 128
NJ = BPW // IDXC
L = 16
UNROLL = 8


def _sc_body(uid_hbm, pid_hbm, nid_hbm, uemb_hbm, iemb_hbm, ubias_hbm,
             ibias_hbm, gb_hbm, pos_out, neg_out,
             uid_v, pid_v, nid_v, ue_v, iep_v, ien_v, ub_v, ibp_v, ibn_v,
             gb_v, outp_v, outn_v, sem):
    wid = lax.axis_index("s") * NC + lax.axis_index("c")
    base = wid * BPW

    s0 = wid // 2
    s1 = (wid % 2) * NJ
    pltpu.sync_copy(uid_hbm.at[s0, pl.ds(s1, NJ)], uid_v)
    pltpu.sync_copy(pid_hbm.at[s0, pl.ds(s1, NJ)], pid_v)
    pltpu.sync_copy(nid_hbm.at[s0, pl.ds(s1, NJ)], nid_v)
    pltpu.sync_copy(gb_hbm, gb_v)

    copies = []
    for j in range(NJ):
        rows = pl.ds(j * IDXC, IDXC)
        copies.append(pltpu.async_copy(uemb_hbm.at[uid_v.at[j]], ue_v.at[rows], sem))
        copies.append(pltpu.async_copy(iemb_hbm.at[pid_v.at[j]], iep_v.at[rows], sem))
        copies.append(pltpu.async_copy(iemb_hbm.at[nid_v.at[j]], ien_v.at[rows], sem))
        copies.append(pltpu.async_copy(ubias_hbm.at[uid_v.at[j]], ub_v.at[rows], sem))
        copies.append(pltpu.async_copy(ibias_hbm.at[pid_v.at[j]], ibp_v.at[rows], sem))
        copies.append(pltpu.async_copy(ibias_hbm.at[nid_v.at[j]], ibn_v.at[rows], sem))
    for c in copies:
        c.wait()

    def dot_body(i, carry):
        for k in range(UNROLL):
            e = i * UNROLL + k
            ue = ue_v[e]
            iep_v[e] = plsc.cumsum(ue * iep_v[e])
            ien_v[e] = plsc.cumsum(ue * ien_v[e])
        return carry

    lax.fori_loop(0, BPW // UNROLL, dot_body, 0)

    lane = lax.iota(jnp.int32, L)
    col15 = jnp.full((L,), EMBED_DIM - 1, jnp.int32)
    gb = gb_v[...]
    for c in range(BPW // L):
        s = pl.ds(c * L, L)
        rows = lane + (c * L)
        dp = plsc.load_gather(iep_v, [rows, col15])
        dn = plsc.load_gather(ien_v, [rows, col15])
        ub = ub_v[s] + gb
        outp_v[s] = dp + (ub + ibp_v[s])
        outn_v[s] = dn + (ub + ibn_v[s])

    pltpu.sync_copy(outp_v, pos_out.at[pl.ds(base, BPW)])
    pltpu.sync_copy(outn_v, neg_out.at[pl.ds(base, BPW)])


_sc_call = pl.kernel(
    _sc_body,
    out_type=(
        jax.ShapeDtypeStruct((BATCH,), jnp.float32),
        jax.ShapeDtypeStruct((BATCH,), jnp.float32),
    ),
    mesh=plsc.VectorSubcoreMesh(core_axis_name="c", subcore_axis_name="s",
                                num_cores=NC, num_subcores=NS),
    compiler_params=pltpu.CompilerParams(needs_layout_passes=False,
                                         use_tc_tiling_on_sc=False),
    scratch_types=[
        pltpu.VMEM((NJ, IDXC), jnp.int32),
        pltpu.VMEM((NJ, IDXC), jnp.int32),
        pltpu.VMEM((NJ, IDXC), jnp.int32),
        pltpu.VMEM((BPW, EMBED_DIM), jnp.float32),
        pltpu.VMEM((BPW, EMBED_DIM), jnp.float32),
        pltpu.VMEM((BPW, EMBED_DIM), jnp.float32),
        pltpu.VMEM((BPW,), jnp.float32),
        pltpu.VMEM((BPW,), jnp.float32),
        pltpu.VMEM((BPW,), jnp.float32),
        pltpu.VMEM((L,), jnp.float32),
        pltpu.VMEM((BPW,), jnp.float32),
        pltpu.VMEM((BPW,), jnp.float32),
        pltpu.SemaphoreType.DMA,
    ],
)


def kernel(user_ids, pos_item_ids, neg_item_ids, user_emb, item_emb,
           user_bias, item_bias, global_bias):
    uid = user_ids.astype(jnp.int32).reshape(BATCH // 1024, 8, 128)
    pid = pos_item_ids.astype(jnp.int32).reshape(BATCH // 1024, 8, 128)
    nid = neg_item_ids.astype(jnp.int32).reshape(BATCH // 1024, 8, 128)
    ub = jnp.sum(user_bias, axis=1)
    ib = jnp.sum(item_bias, axis=1)
    gb = jnp.broadcast_to(global_bias, (L,))
    return _sc_call(uid, pid, nid, user_emb, item_emb, ub, ib, gb)
